# Initial kernel scaffold; baseline (speedup 1.0000x reference)
import functools

import jax
import jax.numpy as jnp
from jax import lax
from jax.experimental import pallas as pl
from jax.experimental.pallas import tpu as pltpu
from jax.experimental.pallas import tpu_sc as plsc

N = 10000
E = 320000
D1 = 128
DO = 40
DOP = 48

NC, NS, L = 2, 16, 16
NW = NC * NS
EW = E // NW
K = 80
NCH = EW // K
RPT = N // NS

R = 1000
GRID = N // R

_f32 = jnp.float32
_mesh = plsc.VectorSubcoreMesh(core_axis_name="c", subcore_axis_name="s")


@functools.partial(
    pl.kernel,
    out_type=jax.ShapeDtypeStruct((NC, N), _f32),
    mesh=_mesh,
    scratch_types=[
        pltpu.VMEM((NCH, K), jnp.int32),
        pltpu.VMEM((K,), _f32),
        pltpu.VMEM((N,), _f32),
        pltpu.VMEM_SHARED((N,), _f32),
    ],
)
def _deg_kernel(dst_hbm, out_hbm, idx_v, ones_v, zbuf_v, acc_sh):
    c = lax.axis_index("c")
    s = lax.axis_index("s")
    w = c * NS + s

    @pl.loop(0, K // L)
    def _(i):
        ones_v[pl.ds(i * L, L)] = jnp.ones((L,), _f32)

    @pl.when(s == 0)
    def _():
        @pl.loop(0, N // L)
        def _(i):
            zbuf_v[pl.ds(i * L, L)] = jnp.zeros((L,), _f32)

        pltpu.sync_copy(zbuf_v, acc_sh)

    pltpu.sync_copy(dst_hbm.at[pl.ds(w * NCH, NCH), :], idx_v)
    plsc.subcore_barrier()

    @pl.loop(0, NCH)
    def _(j):
        pltpu.sync_copy(ones_v, acc_sh.at[idx_v.at[j]], add=True)

    plsc.subcore_barrier()

    @pl.when(s == 0)
    def _():
        pltpu.sync_copy(acc_sh, out_hbm.at[c])


def _make_agg(D):
    zrows = 125

    @functools.partial(
        pl.kernel,
        out_type=jax.ShapeDtypeStruct((NC, N, D), _f32),
        mesh=_mesh,
        scratch_types=[
            pltpu.VMEM((NCH, K), jnp.int32),
            pltpu.VMEM((NCH, K), jnp.int32),
            pltpu.VMEM((K, D), _f32),
            pltpu.VMEM((K, D), _f32),
            pltpu.VMEM((zrows, D), _f32),
            pltpu.VMEM_SHARED((N, D), _f32),
            pltpu.SemaphoreType.DMA,
            pltpu.SemaphoreType.DMA,
        ],
    )
    def agg(g_hbm, src_hbm, dst_hbm, out_hbm,
            sidx, didx, buf0, buf1, zbuf, acc_sh, sem0, sem1):
        c = lax.axis_index("c")
        s = lax.axis_index("s")
        w = c * NS + s

        @pl.loop(0, zrows)
        def _(r):
            for cc in range(D // L):
                zbuf[r, pl.ds(cc * L, L)] = jnp.zeros((L,), _f32)

        for t in range(RPT // zrows):
            pltpu.sync_copy(zbuf, acc_sh.at[pl.ds(s * RPT + t * zrows, zrows), :])

        pltpu.sync_copy(src_hbm.at[pl.ds(w * NCH, NCH), :], sidx)
        pltpu.sync_copy(dst_hbm.at[pl.ds(w * NCH, NCH), :], didx)
        plsc.subcore_barrier()

        pltpu.async_copy(g_hbm.at[sidx.at[0]], buf0, sem0)

        @pl.loop(0, NCH - 1, step=2)
        def _(j):
            pltpu.make_async_copy(g_hbm.at[sidx.at[j]], buf0, sem0).wait()
            pltpu.async_copy(g_hbm.at[sidx.at[j + 1]], buf1, sem1)
            pltpu.sync_copy(buf0, acc_sh.at[didx.at[j]], add=True)
            pltpu.make_async_copy(g_hbm.at[sidx.at[j]], buf1, sem1).wait()
            pltpu.async_copy(g_hbm.at[sidx.at[j + 2]], buf0, sem0)
            pltpu.sync_copy(buf1, acc_sh.at[didx.at[j + 1]], add=True)

        pltpu.make_async_copy(g_hbm.at[sidx.at[0]], buf0, sem0).wait()
        pltpu.sync_copy(buf0, acc_sh.at[didx.at[NCH - 1]], add=True)

        plsc.subcore_barrier()
        pltpu.sync_copy(acc_sh.at[pl.ds(s * RPT, RPT), :],
                        out_hbm.at[c, pl.ds(s * RPT, RPT), :])

    return agg


_agg128 = _make_agg(D1)
_agg48 = _make_agg(DOP)


def _dinv_of(degT):
    d = degT[:, 0:1] + degT[:, 1:2] + 1.0
    return lax.rsqrt(d)


def _g1_body(x_ref, w1_ref, degT_ref, o_ref):
    dinv = _dinv_of(degT_ref[...])
    o_ref[...] = jnp.dot(x_ref[...], w1_ref[...],
                         preferred_element_type=_f32) * dinv


_g1_call = pl.pallas_call(
    _g1_body,
    grid=(GRID,),
    in_specs=[
        pl.BlockSpec((R, D1), lambda i: (i, 0)),
        pl.BlockSpec((D1, D1), lambda i: (0, 0)),
        pl.BlockSpec((R, NC), lambda i: (i, 0)),
    ],
    out_specs=pl.BlockSpec((R, D1), lambda i: (i, 0)),
    out_shape=jax.ShapeDtypeStruct((N, D1), _f32),
)


def _mid_body(s1a_ref, s1b_ref, g1_ref, degT_ref, w2_ref, b1_ref, o_ref):
    dinv = _dinv_of(degT_ref[...])
    z = dinv * (s1a_ref[...] + s1b_ref[...] + g1_ref[...]) + b1_ref[...]
    z = jnp.maximum(z, 0.0)
    o_ref[...] = jnp.dot(z, w2_ref[...], preferred_element_type=_f32) * dinv


_mid_call = pl.pallas_call(
    _mid_body,
    grid=(GRID,),
    in_specs=[
        pl.BlockSpec((R, D1), lambda i: (i, 0)),
        pl.BlockSpec((R, D1), lambda i: (i, 0)),
        pl.BlockSpec((R, D1), lambda i: (i, 0)),
        pl.BlockSpec((R, NC), lambda i: (i, 0)),
        pl.BlockSpec((D1, DOP), lambda i: (0, 0)),
        pl.BlockSpec((1, D1), lambda i: (0, 0)),
    ],
    out_specs=pl.BlockSpec((R, DOP), lambda i: (i, 0)),
    out_shape=jax.ShapeDtypeStruct((N, DOP), _f32),
)


def _final_body(s2a_ref, s2b_ref, g2_ref, degT_ref, b2_ref, o_ref):
    dinv = _dinv_of(degT_ref[...])
    u = dinv * (s2a_ref[...] + s2b_ref[...] + g2_ref[...]) + b2_ref[...]
    col = lax.broadcasted_iota(jnp.int32, (R, DOP), 1)
    valid = col < DO
    um = jnp.where(valid, u, -jnp.inf)
    m = jnp.max(um, axis=1, keepdims=True)
    ex = jnp.where(valid, jnp.exp(u - m), 0.0)
    lse = jnp.log(jnp.sum(ex, axis=1, keepdims=True))
    o_ref[...] = (u - m - lse)[:, :DO]


_final_call = pl.pallas_call(
    _final_body,
    grid=(GRID,),
    in_specs=[
        pl.BlockSpec((R, DOP), lambda i: (i, 0)),
        pl.BlockSpec((R, DOP), lambda i: (i, 0)),
        pl.BlockSpec((R, DOP), lambda i: (i, 0)),
        pl.BlockSpec((R, NC), lambda i: (i, 0)),
        pl.BlockSpec((1, DOP), lambda i: (0, 0)),
    ],
    out_specs=pl.BlockSpec((R, DO), lambda i: (i, 0)),
    out_shape=jax.ShapeDtypeStruct((N, DO), _f32),
)


def kernel(x, edge_index, W1, b1, W2, b2):
    ei = edge_index.astype(jnp.int32)
    src2 = ei[0].reshape(E // K, K)
    dst2 = ei[1].reshape(E // K, K)

    degp = _deg_kernel(dst2)
    degT = jnp.transpose(degp)

    g1 = _g1_call(x, W1, degT)
    s1 = _agg128(g1, src2, dst2)

    W2p = jnp.pad(W2, ((0, 0), (0, DOP - DO)))
    b2p = jnp.pad(b2, (0, DOP - DO))
    g2 = _mid_call(s1[0], s1[1], g1, degT, W2p, b1.reshape(1, D1))
    s2 = _agg48(g2, src2, dst2)

    return _final_call(s2[0], s2[1], g2, degT, b2p.reshape(1, DOP))


# speedup vs baseline: 20.8535x; 20.8535x over previous
import functools

import jax
import jax.numpy as jnp
from jax import lax
from jax.experimental import pallas as pl
from jax.experimental.pallas import tpu as pltpu
from jax.experimental.pallas import tpu_sc as plsc

N = 10000
E = 320000
D1 = 128
DH = 64
DO = 40
DOP = 48

NC, NS, L = 2, 16, 16
NW = NC * NS
EW = E // NW
K = 80
NCH = EW // K
NP = 10240
RPT = NP // NS

R = 1000
GRID = N // R

_f32 = jnp.float32
_SC_PARAMS = pltpu.CompilerParams(use_tc_tiling_on_sc=False)


@functools.cache
def _make_deg():
    mesh = plsc.VectorSubcoreMesh(core_axis_name="c", subcore_axis_name="s")

    @functools.partial(
        pl.kernel,
        out_type=jax.ShapeDtypeStruct((NC, NP), _f32),
        mesh=mesh,
        compiler_params=_SC_PARAMS,
        scratch_types=[
            pltpu.VMEM((NCH, K), jnp.int32),
            pltpu.VMEM((K,), _f32),
            pltpu.VMEM((NP,), _f32),
            pltpu.VMEM_SHARED((NP,), _f32),
        ],
    )
    def deg_kernel(dst_hbm, out_hbm, idx_v, ones_v, zbuf_v, acc_sh):
        c = lax.axis_index("c")
        s = lax.axis_index("s")
        w = c * NS + s

        @pl.loop(0, K // L)
        def _(i):
            ones_v[pl.ds(i * L, L)] = jnp.ones((L,), _f32)

        @pl.when(s == 0)
        def _():
            @pl.loop(0, NP // L)
            def _(i):
                zbuf_v[pl.ds(i * L, L)] = jnp.zeros((L,), _f32)

            pltpu.sync_copy(zbuf_v, acc_sh)

        pltpu.sync_copy(dst_hbm.at[w], idx_v)
        plsc.subcore_barrier()

        @pl.loop(0, NCH)
        def _(j):
            pltpu.sync_copy(ones_v, acc_sh.at[idx_v.at[j]], add=True)

        plsc.subcore_barrier()

        @pl.when(s == 0)
        def _():
            pltpu.sync_copy(acc_sh, out_hbm.at[c])

    return deg_kernel


@functools.cache
def _make_agg(D):
    zrows = 128
    mesh = plsc.VectorSubcoreMesh(core_axis_name="c", subcore_axis_name="s")

    @functools.partial(
        pl.kernel,
        out_type=jax.ShapeDtypeStruct((NC, NP, D), _f32),
        mesh=mesh,
        compiler_params=_SC_PARAMS,
        scratch_types=[
            pltpu.VMEM((NCH, K), jnp.int32),
            pltpu.VMEM((NCH, K), jnp.int32),
            pltpu.VMEM((K, D), _f32),
            pltpu.VMEM((K, D), _f32),
            pltpu.VMEM((zrows, D), _f32),
            pltpu.VMEM_SHARED((NP, D), _f32),
            pltpu.SemaphoreType.DMA,
            pltpu.SemaphoreType.DMA,
        ],
    )
    def agg(g_hbm, src_hbm, dst_hbm, out_hbm,
            sidx, didx, buf0, buf1, zbuf, acc_sh, sem0, sem1):
        c = lax.axis_index("c")
        s = lax.axis_index("s")
        w = c * NS + s

        @pl.loop(0, zrows)
        def _(r):
            for cc in range(D // L):
                zbuf[r, pl.ds(cc * L, L)] = jnp.zeros((L,), _f32)

        for t in range(RPT // zrows):
            pltpu.sync_copy(zbuf,
                            acc_sh.at[pl.ds(s * RPT + t * zrows, zrows), :])

        pltpu.sync_copy(src_hbm.at[w], sidx)
        pltpu.sync_copy(dst_hbm.at[w], didx)
        plsc.subcore_barrier()

        pltpu.async_copy(g_hbm.at[sidx.at[0]], buf0, sem0)

        @pl.loop(0, NCH - 1, step=2)
        def _(j):
            pltpu.make_async_copy(g_hbm.at[sidx.at[j]], buf0, sem0).wait()
            pltpu.async_copy(g_hbm.at[sidx.at[j + 1]], buf1, sem1)
            pltpu.sync_copy(buf0, acc_sh.at[didx.at[j]], add=True)
            pltpu.make_async_copy(g_hbm.at[sidx.at[j]], buf1, sem1).wait()
            pltpu.async_copy(g_hbm.at[sidx.at[j + 2]], buf0, sem0)
            pltpu.sync_copy(buf1, acc_sh.at[didx.at[j + 1]], add=True)

        pltpu.make_async_copy(g_hbm.at[sidx.at[0]], buf0, sem0).wait()
        pltpu.sync_copy(buf0, acc_sh.at[didx.at[NCH - 1]], add=True)

        plsc.subcore_barrier()
        pltpu.sync_copy(acc_sh.at[pl.ds(s * RPT, RPT), :],
                        out_hbm.at[c, pl.ds(s * RPT, RPT), :])

    return agg


def _dinv_of(degT):
    d = degT[:, 0:1] + degT[:, 1:2] + 1.0
    return lax.rsqrt(d)


def _g1_body(x_ref, w1_ref, degT_ref, oa_ref, ob_ref):
    dinv = _dinv_of(degT_ref[...])
    g = jnp.dot(x_ref[...], w1_ref[...], preferred_element_type=_f32) * dinv
    oa_ref[...] = g[:, :DH]
    ob_ref[...] = g[:, DH:]


_g1_call = pl.pallas_call(
    _g1_body,
    grid=(GRID,),
    in_specs=[
        pl.BlockSpec((R, D1), lambda i: (i, 0)),
        pl.BlockSpec((D1, D1), lambda i: (0, 0)),
        pl.BlockSpec((R, NC), lambda i: (i, 0)),
    ],
    out_specs=[
        pl.BlockSpec((R, DH), lambda i: (i, 0)),
        pl.BlockSpec((R, DH), lambda i: (i, 0)),
    ],
    out_shape=[
        jax.ShapeDtypeStruct((N, DH), _f32),
        jax.ShapeDtypeStruct((N, DH), _f32),
    ],
)


def _mid_body(s1al_ref, s1bl_ref, s1ar_ref, s1br_ref, g1l_ref, g1r_ref,
              degT_ref, w2_ref, b1_ref, o_ref):
    dinv = _dinv_of(degT_ref[...])
    zl = s1al_ref[...] + s1bl_ref[...] + g1l_ref[...]
    zr = s1ar_ref[...] + s1br_ref[...] + g1r_ref[...]
    z = dinv * jnp.concatenate([zl, zr], axis=1) + b1_ref[...]
    z = jnp.maximum(z, 0.0)
    o_ref[...] = jnp.dot(z, w2_ref[...], preferred_element_type=_f32) * dinv


_mid_call = pl.pallas_call(
    _mid_body,
    grid=(GRID,),
    in_specs=[
        pl.BlockSpec((R, DH), lambda i: (i, 0)),
        pl.BlockSpec((R, DH), lambda i: (i, 0)),
        pl.BlockSpec((R, DH), lambda i: (i, 0)),
        pl.BlockSpec((R, DH), lambda i: (i, 0)),
        pl.BlockSpec((R, DH), lambda i: (i, 0)),
        pl.BlockSpec((R, DH), lambda i: (i, 0)),
        pl.BlockSpec((R, NC), lambda i: (i, 0)),
        pl.BlockSpec((D1, DOP), lambda i: (0, 0)),
        pl.BlockSpec((1, D1), lambda i: (0, 0)),
    ],
    out_specs=pl.BlockSpec((R, DOP), lambda i: (i, 0)),
    out_shape=jax.ShapeDtypeStruct((N, DOP), _f32),
)


def _final_body(s2a_ref, s2b_ref, g2_ref, degT_ref, b2_ref, o_ref):
    dinv = _dinv_of(degT_ref[...])
    u = dinv * (s2a_ref[...] + s2b_ref[...] + g2_ref[...]) + b2_ref[...]
    col = lax.broadcasted_iota(jnp.int32, (R, DOP), 1)
    valid = col < DO
    um = jnp.where(valid, u, -jnp.inf)
    m = jnp.max(um, axis=1, keepdims=True)
    ex = jnp.where(valid, jnp.exp(u - m), 0.0)
    lse = jnp.log(jnp.sum(ex, axis=1, keepdims=True))
    o_ref[...] = (u - m - lse)[:, :DO]


_final_call = pl.pallas_call(
    _final_body,
    grid=(GRID,),
    in_specs=[
        pl.BlockSpec((R, DOP), lambda i: (i, 0)),
        pl.BlockSpec((R, DOP), lambda i: (i, 0)),
        pl.BlockSpec((R, DOP), lambda i: (i, 0)),
        pl.BlockSpec((R, NC), lambda i: (i, 0)),
        pl.BlockSpec((1, DOP), lambda i: (0, 0)),
    ],
    out_specs=pl.BlockSpec((R, DO), lambda i: (i, 0)),
    out_shape=jax.ShapeDtypeStruct((N, DO), _f32),
)


def kernel(x, edge_index, W1, b1, W2, b2):
    ei = edge_index.astype(jnp.int32)
    src2 = ei[0].reshape(NW, NCH, K)
    dst2 = ei[1].reshape(NW, NCH, K)

    degp = _make_deg()(dst2)
    degT = jnp.transpose(degp)[:N]

    g1l, g1r = _g1_call(x, W1, degT)
    s1l = _make_agg(DH)(g1l, src2, dst2)
    s1r = _make_agg(DH)(g1r, src2, dst2)

    W2p = jnp.pad(W2, ((0, 0), (0, DOP - DO)))
    b2p = jnp.pad(b2, (0, DOP - DO))
    g2 = _mid_call(s1l[0, :N], s1l[1, :N], s1r[0, :N], s1r[1, :N],
                   g1l, g1r, degT, W2p, b1.reshape(1, D1))
    s2 = _make_agg(DOP)(g2, src2, dst2)

    return _final_call(s2[0, :N], s2[1, :N], g2, degT, b2p.reshape(1, DOP))


# speedup vs baseline: 34.6429x; 1.6612x over previous
import functools

import jax
import jax.numpy as jnp
from jax import lax
from jax.experimental import pallas as pl
from jax.experimental.pallas import tpu as pltpu
from jax.experimental.pallas import tpu_sc as plsc

N = 10000
E = 320000
D1 = 128
DH = 64
DO = 40
DOP = 48

NC, NS, L = 2, 16, 16
NW = NC * NS
EW = E // NW
K = 125
NCH = EW // K
NBUF = 4
NP = 10240
RPT = NP // NS

R = 1000
GRID = N // R

_f32 = jnp.float32
_SC_PARAMS = pltpu.CompilerParams(use_tc_tiling_on_sc=False)


@functools.cache
def _make_deg():
    mesh = plsc.VectorSubcoreMesh(core_axis_name="c", subcore_axis_name="s")

    @functools.partial(
        pl.kernel,
        out_type=jax.ShapeDtypeStruct((NC, NP), _f32),
        mesh=mesh,
        compiler_params=_SC_PARAMS,
        scratch_types=[
            pltpu.VMEM((NCH, K), jnp.int32),
            pltpu.VMEM((128,), _f32),
            pltpu.VMEM((NP,), _f32),
            pltpu.VMEM_SHARED((NP,), _f32),
        ],
    )
    def deg_kernel(dst_hbm, out_hbm, idx_v, ones_v, zbuf_v, acc_sh):
        c = lax.axis_index("c")
        s = lax.axis_index("s")
        w = c * NS + s

        @pl.loop(0, 128 // L)
        def _(i):
            ones_v[pl.ds(i * L, L)] = jnp.ones((L,), _f32)

        @pl.when(s == 0)
        def _():
            @pl.loop(0, NP // L)
            def _(i):
                zbuf_v[pl.ds(i * L, L)] = jnp.zeros((L,), _f32)

            pltpu.sync_copy(zbuf_v, acc_sh)

        pltpu.sync_copy(dst_hbm.at[w], idx_v)
        plsc.subcore_barrier()

        @pl.loop(0, NCH)
        def _(j):
            pltpu.sync_copy(ones_v.at[pl.ds(0, K)], acc_sh.at[idx_v.at[j]],
                            add=True)

        plsc.subcore_barrier()

        @pl.when(s == 0)
        def _():
            pltpu.sync_copy(acc_sh, out_hbm.at[c])

    return deg_kernel


@functools.cache
def _make_agg(D, nphase):
    zrows = 128
    mesh = plsc.VectorSubcoreMesh(core_axis_name="c", subcore_axis_name="s")

    @functools.partial(
        pl.kernel,
        out_type=jax.ShapeDtypeStruct((nphase, NC, NP, D), _f32),
        mesh=mesh,
        compiler_params=_SC_PARAMS,
        scratch_types=[
            pltpu.VMEM((NCH, K), jnp.int32),
            pltpu.VMEM((NCH, K), jnp.int32),
            [pltpu.VMEM((K, D), _f32)] * NBUF,
            pltpu.VMEM((zrows, D), _f32),
            pltpu.VMEM_SHARED((NP, D), _f32),
            [pltpu.SemaphoreType.DMA] * NBUF,
            pltpu.SemaphoreType.DMA,
        ],
    )
    def agg(*refs):
        g_hbms = refs[:nphase]
        src_hbm, dst_hbm, out_hbm = refs[nphase:nphase + 3]
        sidx, didx, bufs, zbuf, acc_sh, gsems, isem = refs[nphase + 3:]
        c = lax.axis_index("c")
        s = lax.axis_index("s")
        w = c * NS + s

        pltpu.async_copy(src_hbm.at[w], sidx, isem)
        pltpu.async_copy(dst_hbm.at[w], didx, isem)

        @pl.loop(0, zrows)
        def _(r):
            for cc in range(D // L):
                zbuf[r, pl.ds(cc * L, L)] = jnp.zeros((L,), _f32)

        pltpu.make_async_copy(src_hbm.at[w], sidx, isem).wait()
        pltpu.make_async_copy(dst_hbm.at[w], didx, isem).wait()

        for p in range(nphase):
            g_hbm = g_hbms[p]

            for t in range(RPT // zrows):
                pltpu.sync_copy(
                    zbuf, acc_sh.at[pl.ds(s * RPT + t * zrows, zrows), :])
            plsc.subcore_barrier()

            for b in range(NBUF):
                pltpu.async_copy(g_hbm.at[sidx.at[b]], bufs[b], gsems[b])

            @pl.loop(0, NCH, step=NBUF)
            def _(jj):
                for b in range(NBUF):
                    j = jj + b
                    pltpu.make_async_copy(
                        g_hbm.at[sidx.at[j]], bufs[b], gsems[b]).wait()
                    pltpu.sync_copy(bufs[b], acc_sh.at[didx.at[j]], add=True)

                    @pl.when(j + NBUF < NCH)
                    def _():
                        pltpu.async_copy(
                            g_hbm.at[sidx.at[j + NBUF]], bufs[b], gsems[b])

            plsc.subcore_barrier()
            pltpu.sync_copy(acc_sh.at[pl.ds(s * RPT, RPT), :],
                            out_hbm.at[p, c, pl.ds(s * RPT, RPT), :])
            if p + 1 < nphase:
                plsc.subcore_barrier()

    return agg


def _dinv_of(degT):
    d = degT[:, 0:1] + degT[:, 1:2] + 1.0
    return lax.rsqrt(d)


def _g1_body(x_ref, w1_ref, degT_ref, oa_ref, ob_ref):
    dinv = _dinv_of(degT_ref[...])
    g = jnp.dot(x_ref[...], w1_ref[...], preferred_element_type=_f32) * dinv
    oa_ref[...] = g[:, :DH]
    ob_ref[...] = g[:, DH:]


_g1_call = pl.pallas_call(
    _g1_body,
    grid=(GRID,),
    in_specs=[
        pl.BlockSpec((R, D1), lambda i: (i, 0)),
        pl.BlockSpec((D1, D1), lambda i: (0, 0)),
        pl.BlockSpec((R, NC), lambda i: (i, 0)),
    ],
    out_specs=[
        pl.BlockSpec((R, DH), lambda i: (i, 0)),
        pl.BlockSpec((R, DH), lambda i: (i, 0)),
    ],
    out_shape=[
        jax.ShapeDtypeStruct((N, DH), _f32),
        jax.ShapeDtypeStruct((N, DH), _f32),
    ],
)


def _mid_body(s1al_ref, s1bl_ref, s1ar_ref, s1br_ref, g1l_ref, g1r_ref,
              degT_ref, w2_ref, b1_ref, o_ref):
    dinv = _dinv_of(degT_ref[...])
    zl = s1al_ref[...] + s1bl_ref[...] + g1l_ref[...]
    zr = s1ar_ref[...] + s1br_ref[...] + g1r_ref[...]
    z = dinv * jnp.concatenate([zl, zr], axis=1) + b1_ref[...]
    z = jnp.maximum(z, 0.0)
    o_ref[...] = jnp.dot(z, w2_ref[...], preferred_element_type=_f32) * dinv


_mid_call = pl.pallas_call(
    _mid_body,
    grid=(GRID,),
    in_specs=[
        pl.BlockSpec((R, DH), lambda i: (i, 0)),
        pl.BlockSpec((R, DH), lambda i: (i, 0)),
        pl.BlockSpec((R, DH), lambda i: (i, 0)),
        pl.BlockSpec((R, DH), lambda i: (i, 0)),
        pl.BlockSpec((R, DH), lambda i: (i, 0)),
        pl.BlockSpec((R, DH), lambda i: (i, 0)),
        pl.BlockSpec((R, NC), lambda i: (i, 0)),
        pl.BlockSpec((D1, DOP), lambda i: (0, 0)),
        pl.BlockSpec((1, D1), lambda i: (0, 0)),
    ],
    out_specs=pl.BlockSpec((R, DOP), lambda i: (i, 0)),
    out_shape=jax.ShapeDtypeStruct((N, DOP), _f32),
)


def _final_body(s2a_ref, s2b_ref, g2_ref, degT_ref, b2_ref, o_ref):
    dinv = _dinv_of(degT_ref[...])
    u = dinv * (s2a_ref[...] + s2b_ref[...] + g2_ref[...]) + b2_ref[...]
    col = lax.broadcasted_iota(jnp.int32, (R, DOP), 1)
    valid = col < DO
    um = jnp.where(valid, u, -jnp.inf)
    m = jnp.max(um, axis=1, keepdims=True)
    ex = jnp.where(valid, jnp.exp(u - m), 0.0)
    lse = jnp.log(jnp.sum(ex, axis=1, keepdims=True))
    o_ref[...] = (u - m - lse)[:, :DO]


_final_call = pl.pallas_call(
    _final_body,
    grid=(GRID,),
    in_specs=[
        pl.BlockSpec((R, DOP), lambda i: (i, 0)),
        pl.BlockSpec((R, DOP), lambda i: (i, 0)),
        pl.BlockSpec((R, DOP), lambda i: (i, 0)),
        pl.BlockSpec((R, NC), lambda i: (i, 0)),
        pl.BlockSpec((1, DOP), lambda i: (0, 0)),
    ],
    out_specs=pl.BlockSpec((R, DO), lambda i: (i, 0)),
    out_shape=jax.ShapeDtypeStruct((N, DO), _f32),
)


def kernel(x, edge_index, W1, b1, W2, b2):
    ei = edge_index.astype(jnp.int32)
    src2 = ei[0].reshape(NW, NCH, K)
    dst2 = ei[1].reshape(NW, NCH, K)

    degp = _make_deg()(dst2)
    degT = jnp.transpose(degp)[:N]

    g1l, g1r = _g1_call(x, W1, degT)
    s1 = _make_agg(DH, 2)(g1l, g1r, src2, dst2)

    W2p = jnp.pad(W2, ((0, 0), (0, DOP - DO)))
    b2p = jnp.pad(b2, (0, DOP - DO))
    g2 = _mid_call(s1[0, 0, :N], s1[0, 1, :N], s1[1, 0, :N], s1[1, 1, :N],
                   g1l, g1r, degT, W2p, b1.reshape(1, D1))
    s2 = _make_agg(DOP, 1)(g2, src2, dst2)

    return _final_call(s2[0, 0, :N], s2[0, 1, :N], g2, degT,
                       b2p.reshape(1, DOP))
